# Initial kernel scaffold; baseline (speedup 1.0000x reference)
import jax
import jax.numpy as jnp
from jax.experimental import pallas as pl


def kernel(x, edge_index, batch, c1_Wl, c1_bl, c1_Wr, c2_Wl, c2_bl, c2_Wr, c3_Wl, c3_bl, c3_Wr, p1_Wrel, p1_brel, p1_Wroot, p2_Wrel, p2_brel, p2_Wroot, p3_Wrel, p3_brel, p3_Wroot, lin1_W, lin1_b, lin2_W, lin2_b, lin3_W, lin3_b):
    raise NotImplementedError("write your pallas kernel here")


# speedup vs baseline: 1.1056x; 1.1056x over previous
import functools
import numpy as np
import jax
import jax.numpy as jnp
from jax.experimental import pallas as pl
from jax.experimental.pallas import tpu as pltpu

_RATIO = 0.5


def _conv_block(x_ref, s_ref, cnt_ref, wlT_ref, wrT_ref, bl_ref, h_ref):
    cnt = jnp.maximum(cnt_ref[...], 1.0)
    mean = s_ref[...] / cnt
    t = (jnp.dot(mean, wlT_ref[...], preferred_element_type=jnp.float32)
         + jnp.dot(x_ref[...], wrT_ref[...], preferred_element_type=jnp.float32)
         + bl_ref[...])
    h_ref[...] = jnp.where(t >= 0, t, 0.1 * t)


def _conv(x, s, cnt, Wl, bl, Wr):
    n, din = x.shape
    hid = Wl.shape[0]
    R = 1000 if n % 1000 == 0 else _row_block(n)
    h = pl.pallas_call(
        _conv_block,
        grid=(n // R,),
        in_specs=[
            pl.BlockSpec((R, din), lambda i: (i, 0)),
            pl.BlockSpec((R, din), lambda i: (i, 0)),
            pl.BlockSpec((R, 1), lambda i: (i, 0)),
            pl.BlockSpec((din, hid), lambda i: (0, 0)),
            pl.BlockSpec((din, hid), lambda i: (0, 0)),
            pl.BlockSpec((1, hid), lambda i: (0, 0)),
        ],
        out_specs=pl.BlockSpec((R, hid), lambda i: (i, 0)),
        out_shape=jax.ShapeDtypeStruct((n, hid), jnp.float32),
    )(x, s, cnt.reshape(-1, 1), Wl.T, Wr.T, bl.reshape(1, -1))
    return h


def _row_block(n):
    for r in (1000, 500, 250, 1250, 625, 200, 100, 40, 8):
        if n % r == 0 and (r % 8 == 0 or r == n):
            return r
    return n


def _seg_sum(data, seg, n):
    return jnp.zeros((n,) + data.shape[1:], data.dtype).at[seg].add(data)


def kernel(x, edge_index, batch, c1_Wl, c1_bl, c1_Wr, c2_Wl, c2_bl, c2_Wr,
           c3_Wl, c3_bl, c3_Wr, p1_Wrel, p1_brel, p1_Wroot, p2_Wrel, p2_brel,
           p2_Wroot, p3_Wrel, p3_brel, p3_Wroot, lin1_W, lin1_b, lin2_W,
           lin2_b, lin3_W, lin3_b):
    lrelu = lambda t: jnp.where(t >= 0, t, 0.1 * t)
    ei = edge_index
    valid = jnp.ones((ei.shape[1],), dtype=bool)
    h = x
    readouts = []
    convs = [(c1_Wl, c1_bl, c1_Wr), (c2_Wl, c2_bl, c2_Wr), (c3_Wl, c3_bl, c3_Wr)]
    pools = [(p1_Wrel, p1_brel, p1_Wroot), (p2_Wrel, p2_brel, p2_Wroot),
             (p3_Wrel, p3_brel, p3_Wroot)]
    for (Wl, bl, Wr), (Wrel, brel, Wroot) in zip(convs, pools):
        n = h.shape[0]
        src, dst = ei[0], ei[1]
        w = valid.astype(jnp.float32)
        s = _seg_sum(h[src] * w[:, None], dst, n)
        cnt = jnp.zeros((n,), jnp.float32).at[dst].add(w)
        h = _conv(h, s, cnt, Wl, bl, Wr)
        agg = _seg_sum(h[src] * w[:, None], dst, n)
        score = (agg @ Wrel.T + brel + h @ Wroot.T).reshape(-1)
        k = int(np.ceil(_RATIO * n))
        vals, perm = jax.lax.top_k(score, k)
        h = h[perm] * jnp.tanh(vals)[:, None]
        mapping = jnp.full((n,), -1, jnp.int32).at[perm].set(
            jnp.arange(k, dtype=jnp.int32))
        new_ei = mapping[ei]
        valid = valid & (new_ei[0] >= 0) & (new_ei[1] >= 0)
        ei = jnp.where(new_ei >= 0, new_ei, 0)
        readouts.append(jnp.concatenate(
            [jnp.max(h, axis=0, keepdims=True),
             jnp.mean(h, axis=0, keepdims=True)], axis=1))

    n = h.shape[0]
    row, col = ei[0], ei[1]
    w = (valid & (row != col)).astype(jnp.float32)
    deg = jnp.zeros((n,), jnp.float32).at[row].add(w)
    deg_safe = jnp.where(deg > 0, deg, 1.0)
    dinv = jnp.where(deg > 0, jax.lax.rsqrt(deg_safe), 0.0)
    g = dinv[:, None] * h
    agg = _seg_sum(g[row] * w[:, None], col, n)
    info = -dinv[:, None] * agg + h
    node_score = jnp.sum(jnp.abs(info), axis=1)

    x1, x2, x3 = readouts
    gvec = lrelu(x1) + lrelu(x2) + lrelu(x3)
    graph_emb = gvec
    gn = gvec / jnp.maximum(jnp.linalg.norm(gvec, axis=-1, keepdims=True), 1e-12)
    gn = lrelu(gn @ lin1_W.T + lin1_b)
    gn = lrelu(gn @ lin2_W.T + lin2_b)
    gn = gn @ lin3_W.T + lin3_b
    return (gn, jnp.mean(node_score), graph_emb)


# speedup vs baseline: 1.1235x; 1.0162x over previous
import functools
import numpy as np
import jax
import jax.numpy as jnp
from jax import lax
from jax.experimental import pallas as pl
from jax.experimental.pallas import tpu as pltpu

_RATIO = 0.5

_DNUMS_ROW = lax.ScatterDimensionNumbers(
    update_window_dims=(1,), inserted_window_dims=(0,),
    scatter_dims_to_operand_dims=(0,))
_DNUMS_SCAL = lax.ScatterDimensionNumbers(
    update_window_dims=(), inserted_window_dims=(0,),
    scatter_dims_to_operand_dims=(0,))


def _take(a, idx):
    return jnp.take(a, idx, axis=0, mode='clip')


def _sorted_seg_sum_rows(data, seg_sorted, n):
    return lax.scatter_add(
        jnp.zeros((n, data.shape[1]), data.dtype), seg_sorted[:, None], data,
        _DNUMS_ROW, indices_are_sorted=True, unique_indices=False)


def _sorted_seg_sum_scal(vals, seg_sorted, n):
    return lax.scatter_add(
        jnp.zeros((n,), vals.dtype), seg_sorted[:, None], vals,
        _DNUMS_SCAL, indices_are_sorted=True, unique_indices=False)


def _conv_block(x_ref, s_ref, cnt_ref, wlT_ref, wrT_ref, bl_ref, h_ref):
    cnt = jnp.maximum(cnt_ref[...], 1.0)
    mean = s_ref[...] / cnt
    t = (jnp.dot(mean, wlT_ref[...], preferred_element_type=jnp.float32)
         + jnp.dot(x_ref[...], wrT_ref[...], preferred_element_type=jnp.float32)
         + bl_ref[...])
    h_ref[...] = jnp.where(t >= 0, t, 0.1 * t)


def _conv(x, s, cnt, Wl, bl, Wr):
    n, din = x.shape
    hid = Wl.shape[0]
    R = _row_block(n)
    h = pl.pallas_call(
        _conv_block,
        grid=(n // R,),
        in_specs=[
            pl.BlockSpec((R, din), lambda i: (i, 0)),
            pl.BlockSpec((R, din), lambda i: (i, 0)),
            pl.BlockSpec((R, 1), lambda i: (i, 0)),
            pl.BlockSpec((din, hid), lambda i: (0, 0)),
            pl.BlockSpec((din, hid), lambda i: (0, 0)),
            pl.BlockSpec((1, hid), lambda i: (0, 0)),
        ],
        out_specs=pl.BlockSpec((R, hid), lambda i: (i, 0)),
        out_shape=jax.ShapeDtypeStruct((n, hid), jnp.float32),
    )(x, s, cnt.reshape(-1, 1), Wl.T, Wr.T, bl.reshape(1, -1))
    return h


def _row_block(n):
    for r in (1000, 500, 250, 1250, 625, 200, 100, 40, 8):
        if n % r == 0 and (r % 8 == 0 or r == n):
            return r
    return n


def kernel(x, edge_index, batch, c1_Wl, c1_bl, c1_Wr, c2_Wl, c2_bl, c2_Wr,
           c3_Wl, c3_bl, c3_Wr, p1_Wrel, p1_brel, p1_Wroot, p2_Wrel, p2_brel,
           p2_Wroot, p3_Wrel, p3_brel, p3_Wroot, lin1_W, lin1_b, lin2_W,
           lin2_b, lin3_W, lin3_b):
    lrelu = lambda t: jnp.where(t >= 0, t, 0.1 * t)
    ei = edge_index
    valid = jnp.ones((ei.shape[1],), dtype=bool)
    h = x
    readouts = []
    convs = [(c1_Wl, c1_bl, c1_Wr), (c2_Wl, c2_bl, c2_Wr), (c3_Wl, c3_bl, c3_Wr)]
    pools = [(p1_Wrel, p1_brel, p1_Wroot), (p2_Wrel, p2_brel, p2_Wroot),
             (p3_Wrel, p3_brel, p3_Wroot)]
    for (Wl, bl, Wr), (Wrel, brel, Wroot) in zip(convs, pools):
        n = h.shape[0]
        src, dst = ei[0], ei[1]
        w = valid.astype(jnp.float32)
        order = jnp.argsort(dst, stable=True)
        dst_s = _take(dst, order)
        src_s = _take(src, order)
        w_s = _take(w, order)
        hs = _take(h, src_s) * w_s[:, None]
        s = _sorted_seg_sum_rows(hs, dst_s, n)
        cnt = _sorted_seg_sum_scal(w_s, dst_s, n)
        h = _conv(h, s, cnt, Wl, bl, Wr)
        agg = _sorted_seg_sum_rows(_take(h, src_s) * w_s[:, None], dst_s, n)
        score = (agg @ Wrel.T + brel + h @ Wroot.T).reshape(-1)
        k = int(np.ceil(_RATIO * n))
        vals, perm = jax.lax.top_k(score, k)
        h = _take(h, perm) * jnp.tanh(vals)[:, None]
        mapping = jnp.full((n,), -1, jnp.int32).at[perm].set(
            jnp.arange(k, dtype=jnp.int32))
        new_ei = mapping[ei]
        valid = valid & (new_ei[0] >= 0) & (new_ei[1] >= 0)
        ei = jnp.where(new_ei >= 0, new_ei, 0)
        readouts.append(jnp.concatenate(
            [jnp.max(h, axis=0, keepdims=True),
             jnp.mean(h, axis=0, keepdims=True)], axis=1))

    n = h.shape[0]
    row, col = ei[0], ei[1]
    w = (valid & (row != col)).astype(jnp.float32)
    order = jnp.argsort(col, stable=True)
    col_s = _take(col, order)
    row_s = _take(row, order)
    w_s = _take(w, order)
    deg = jnp.zeros((n,), jnp.float32).at[row].add(w)
    deg_safe = jnp.where(deg > 0, deg, 1.0)
    dinv = jnp.where(deg > 0, jax.lax.rsqrt(deg_safe), 0.0)
    g = dinv[:, None] * h
    agg = _sorted_seg_sum_rows(_take(g, row_s) * w_s[:, None], col_s, n)
    info = -dinv[:, None] * agg + h
    node_score = jnp.sum(jnp.abs(info), axis=1)

    x1, x2, x3 = readouts
    gvec = lrelu(x1) + lrelu(x2) + lrelu(x3)
    graph_emb = gvec
    gn = gvec / jnp.maximum(jnp.linalg.norm(gvec, axis=-1, keepdims=True), 1e-12)
    gn = lrelu(gn @ lin1_W.T + lin1_b)
    gn = lrelu(gn @ lin2_W.T + lin2_b)
    gn = gn @ lin3_W.T + lin3_b
    return (gn, jnp.mean(node_score), graph_emb)


# speedup vs baseline: 1.2585x; 1.1202x over previous
import functools
import numpy as np
import jax
import jax.numpy as jnp
from jax import lax
from jax.experimental import pallas as pl
from jax.experimental.pallas import tpu as pltpu

_RATIO = 0.5

_DNUMS_ROW = lax.ScatterDimensionNumbers(
    update_window_dims=(1,), inserted_window_dims=(0,),
    scatter_dims_to_operand_dims=(0,))
_DNUMS_SCAL = lax.ScatterDimensionNumbers(
    update_window_dims=(), inserted_window_dims=(0,),
    scatter_dims_to_operand_dims=(0,))


def _take(a, idx):
    return jnp.take(a, idx, axis=0, mode='clip')


def _sorted_seg_sum_rows(data, seg_sorted, n):
    return lax.scatter_add(
        jnp.zeros((n, data.shape[1]), data.dtype), seg_sorted[:, None], data,
        _DNUMS_ROW, indices_are_sorted=True, unique_indices=False)


def _sorted_seg_sum_scal(vals, seg_sorted, n):
    return lax.scatter_add(
        jnp.zeros((n,), vals.dtype), seg_sorted[:, None], vals,
        _DNUMS_SCAL, indices_are_sorted=True, unique_indices=False)


def _conv_block(x_ref, s_ref, cnt_ref, wlT_ref, wrT_ref, bl_ref, h_ref):
    cnt = jnp.maximum(cnt_ref[...], 1.0)
    mean = s_ref[...] / cnt
    t = (jnp.dot(mean, wlT_ref[...], preferred_element_type=jnp.float32)
         + jnp.dot(x_ref[...], wrT_ref[...], preferred_element_type=jnp.float32)
         + bl_ref[...])
    h_ref[...] = jnp.where(t >= 0, t, 0.1 * t)


def _conv(x, s, cnt, Wl, bl, Wr):
    n, din = x.shape
    hid = Wl.shape[0]
    R = _row_block(n)
    h = pl.pallas_call(
        _conv_block,
        grid=(n // R,),
        in_specs=[
            pl.BlockSpec((R, din), lambda i: (i, 0)),
            pl.BlockSpec((R, din), lambda i: (i, 0)),
            pl.BlockSpec((R, 1), lambda i: (i, 0)),
            pl.BlockSpec((din, hid), lambda i: (0, 0)),
            pl.BlockSpec((din, hid), lambda i: (0, 0)),
            pl.BlockSpec((1, hid), lambda i: (0, 0)),
        ],
        out_specs=pl.BlockSpec((R, hid), lambda i: (i, 0)),
        out_shape=jax.ShapeDtypeStruct((n, hid), jnp.float32),
    )(x, s, cnt.reshape(-1, 1), Wl.T, Wr.T, bl.reshape(1, -1))
    return h


def _row_block(n):
    for r in (1000, 500, 250, 1250, 625, 200, 100, 40, 8):
        if n % r == 0 and (r % 8 == 0 or r == n):
            return r
    return n


def kernel(x, edge_index, batch, c1_Wl, c1_bl, c1_Wr, c2_Wl, c2_bl, c2_Wr,
           c3_Wl, c3_bl, c3_Wr, p1_Wrel, p1_brel, p1_Wroot, p2_Wrel, p2_brel,
           p2_Wroot, p3_Wrel, p3_brel, p3_Wroot, lin1_W, lin1_b, lin2_W,
           lin2_b, lin3_W, lin3_b):
    lrelu = lambda t: jnp.where(t >= 0, t, 0.1 * t)
    ei = edge_index
    valid = jnp.ones((ei.shape[1],), dtype=bool)
    h = x
    readouts = []
    convs = [(c1_Wl, c1_bl, c1_Wr), (c2_Wl, c2_bl, c2_Wr), (c3_Wl, c3_bl, c3_Wr)]
    pools = [(p1_Wrel, p1_brel, p1_Wroot), (p2_Wrel, p2_brel, p2_Wroot),
             (p3_Wrel, p3_brel, p3_Wroot)]
    for (Wl, bl, Wr), (Wrel, brel, Wroot) in zip(convs, pools):
        n = h.shape[0]
        src, dst = ei[0], ei[1]
        w = valid.astype(jnp.float32)
        order = jnp.argsort(dst, stable=True)
        dst_s = _take(dst, order)
        src_s = _take(src, order)
        w_s = _take(w, order)
        hs = _take(h, src_s) * w_s[:, None]
        s = _sorted_seg_sum_rows(hs, dst_s, n)
        cnt = _sorted_seg_sum_scal(w_s, dst_s, n)
        h = _conv(h, s, cnt, Wl, bl, Wr)
        agg = _sorted_seg_sum_rows(_take(h, src_s) * w_s[:, None], dst_s, n)
        score = (agg @ Wrel.T + brel + h @ Wroot.T).reshape(-1)
        k = int(np.ceil(_RATIO * n))
        vals, perm = jax.lax.top_k(score, k)
        h = _take(h, perm) * jnp.tanh(vals)[:, None]
        mapping = jnp.full((n,), -1, jnp.int32).at[perm].set(
            jnp.arange(k, dtype=jnp.int32))
        new_ei = mapping[ei]
        valid = valid & (new_ei[0] >= 0) & (new_ei[1] >= 0)
        ei = jnp.where(new_ei >= 0, new_ei, 0)
        readouts.append(jnp.concatenate(
            [jnp.max(h, axis=0, keepdims=True),
             jnp.mean(h, axis=0, keepdims=True)], axis=1))

    n = h.shape[0]
    row, col = ei[0], ei[1]
    w = (valid & (row != col)).astype(jnp.float32)
    A = jnp.zeros((n, n), jnp.float32).at[row, col].add(w)
    deg = jnp.sum(A, axis=1)
    deg_safe = jnp.where(deg > 0, deg, 1.0)
    dinv = jnp.where(deg > 0, jax.lax.rsqrt(deg_safe), 0.0)
    g = dinv[:, None] * h
    agg = A.T @ g
    info = -dinv[:, None] * agg + h
    node_score = jnp.sum(jnp.abs(info), axis=1)

    x1, x2, x3 = readouts
    gvec = lrelu(x1) + lrelu(x2) + lrelu(x3)
    graph_emb = gvec
    gn = gvec / jnp.maximum(jnp.linalg.norm(gvec, axis=-1, keepdims=True), 1e-12)
    gn = lrelu(gn @ lin1_W.T + lin1_b)
    gn = lrelu(gn @ lin2_W.T + lin2_b)
    gn = gn @ lin3_W.T + lin3_b
    return (gn, jnp.mean(node_score), graph_emb)


# speedup vs baseline: 1.2815x; 1.0183x over previous
import numpy as np
import jax
import jax.numpy as jnp
from jax import lax
from jax.experimental import pallas as pl
from jax.experimental.pallas import tpu as pltpu
from jax.experimental.pallas import tpu_sc as plsc

_RATIO = 0.5
_NW = 32
_GK = 125
_GB = 40


def _sc_gather_body(data_hbm, idx_hbm, out_hbm, idx_v, buf, sem):
    c = lax.axis_index("c")
    s = lax.axis_index("s")
    wid = s * 2 + c
    base = wid * (_GK * _GB)
    pltpu.sync_copy(idx_hbm.at[wid], idx_v)

    def step(j, carry):
        pltpu.async_copy(data_hbm.at[idx_v.at[j]], buf, sem).wait()
        pltpu.sync_copy(buf, out_hbm.at[pl.ds(base + j * _GB, _GB)])
        return carry

    lax.fori_loop(0, _GK, step, 0)


def _sc_gather(data, idx):
    E = idx.shape[0]
    m, D = data.shape
    idx3 = idx.reshape(_NW, _GK, _GB)
    return pl.kernel(
        _sc_gather_body,
        out_type=jax.ShapeDtypeStruct((E, D), jnp.float32),
        mesh=plsc.VectorSubcoreMesh(core_axis_name="c", subcore_axis_name="s"),
        scratch_types=[
            pltpu.VMEM((_GK, _GB), jnp.int32),
            pltpu.VMEM((_GB, D), jnp.float32),
            pltpu.SemaphoreType.DMA,
        ],
    )(data, idx3)

_DNUMS_ROW = lax.ScatterDimensionNumbers(
    update_window_dims=(1,), inserted_window_dims=(0,),
    scatter_dims_to_operand_dims=(0,))
_DNUMS_SCAL = lax.ScatterDimensionNumbers(
    update_window_dims=(), inserted_window_dims=(0,),
    scatter_dims_to_operand_dims=(0,))


def _take(a, idx):
    return jnp.take(a, idx, axis=0, mode='clip')


def _sorted_seg_sum_rows(data, seg_sorted, n):
    return lax.scatter_add(
        jnp.zeros((n, data.shape[1]), data.dtype), seg_sorted[:, None], data,
        _DNUMS_ROW, indices_are_sorted=True, unique_indices=False)


def _sorted_seg_sum_scal(vals, seg_sorted, n):
    return lax.scatter_add(
        jnp.zeros((n,), vals.dtype), seg_sorted[:, None], vals,
        _DNUMS_SCAL, indices_are_sorted=True, unique_indices=False)


def _conv_block(x_ref, s_ref, cnt_ref, wlT_ref, wrT_ref, bl_ref, h_ref):
    cnt = jnp.maximum(cnt_ref[...], 1.0)
    mean = s_ref[...] / cnt
    t = (jnp.dot(mean, wlT_ref[...], preferred_element_type=jnp.float32)
         + jnp.dot(x_ref[...], wrT_ref[...], preferred_element_type=jnp.float32)
         + bl_ref[...])
    h_ref[...] = jnp.where(t >= 0, t, 0.1 * t)


def _conv(x, s, cnt, Wl, bl, Wr):
    n, din = x.shape
    hid = Wl.shape[0]
    R = _row_block(n)
    h = pl.pallas_call(
        _conv_block,
        grid=(n // R,),
        in_specs=[
            pl.BlockSpec((R, din), lambda i: (i, 0)),
            pl.BlockSpec((R, din), lambda i: (i, 0)),
            pl.BlockSpec((R, 1), lambda i: (i, 0)),
            pl.BlockSpec((din, hid), lambda i: (0, 0)),
            pl.BlockSpec((din, hid), lambda i: (0, 0)),
            pl.BlockSpec((1, hid), lambda i: (0, 0)),
        ],
        out_specs=pl.BlockSpec((R, hid), lambda i: (i, 0)),
        out_shape=jax.ShapeDtypeStruct((n, hid), jnp.float32),
    )(x, s, cnt.reshape(-1, 1), Wl.T, Wr.T, bl.reshape(1, -1))
    return h


def _row_block(n):
    for r in (1000, 500, 250, 1250, 625, 200, 100, 40, 8):
        if n % r == 0 and (r % 8 == 0 or r == n):
            return r
    return n


def kernel(x, edge_index, batch, c1_Wl, c1_bl, c1_Wr, c2_Wl, c2_bl, c2_Wr,
           c3_Wl, c3_bl, c3_Wr, p1_Wrel, p1_brel, p1_Wroot, p2_Wrel, p2_brel,
           p2_Wroot, p3_Wrel, p3_brel, p3_Wroot, lin1_W, lin1_b, lin2_W,
           lin2_b, lin3_W, lin3_b):
    lrelu = lambda t: jnp.where(t >= 0, t, 0.1 * t)
    ei = edge_index
    valid = jnp.ones((ei.shape[1],), dtype=bool)
    h = x
    readouts = []
    convs = [(c1_Wl, c1_bl, c1_Wr), (c2_Wl, c2_bl, c2_Wr), (c3_Wl, c3_bl, c3_Wr)]
    pools = [(p1_Wrel, p1_brel, p1_Wroot), (p2_Wrel, p2_brel, p2_Wroot),
             (p3_Wrel, p3_brel, p3_Wroot)]
    for li, ((Wl, bl, Wr), (Wrel, brel, Wroot)) in enumerate(zip(convs, pools)):
        n = h.shape[0]
        src, dst = ei[0], ei[1]
        w = valid.astype(jnp.float32)
        order = jnp.argsort(dst, stable=True)
        dst_s = _take(dst, order)
        src_s = _take(src, order)
        w_s = _take(w, order)
        gather = _sc_gather if li == 0 else _take
        hs = gather(h, src_s) * w_s[:, None]
        s = _sorted_seg_sum_rows(hs, dst_s, n)
        cnt = _sorted_seg_sum_scal(w_s, dst_s, n)
        h = _conv(h, s, cnt, Wl, bl, Wr)
        agg = _sorted_seg_sum_rows(gather(h, src_s) * w_s[:, None], dst_s, n)
        score = (agg @ Wrel.T + brel + h @ Wroot.T).reshape(-1)
        k = int(np.ceil(_RATIO * n))
        vals, perm = jax.lax.top_k(score, k)
        h = _take(h, perm) * jnp.tanh(vals)[:, None]
        mapping = jnp.full((n,), -1, jnp.int32).at[perm].set(
            jnp.arange(k, dtype=jnp.int32))
        new_ei = mapping[ei]
        valid = valid & (new_ei[0] >= 0) & (new_ei[1] >= 0)
        ei = jnp.where(new_ei >= 0, new_ei, 0)
        readouts.append(jnp.concatenate(
            [jnp.max(h, axis=0, keepdims=True),
             jnp.mean(h, axis=0, keepdims=True)], axis=1))

    n = h.shape[0]
    row, col = ei[0], ei[1]
    w = (valid & (row != col)).astype(jnp.float32)
    A = jnp.zeros((n, n), jnp.float32).at[row, col].add(w)
    deg = jnp.sum(A, axis=1)
    deg_safe = jnp.where(deg > 0, deg, 1.0)
    dinv = jnp.where(deg > 0, jax.lax.rsqrt(deg_safe), 0.0)
    g = dinv[:, None] * h
    agg = A.T @ g
    info = -dinv[:, None] * agg + h
    node_score = jnp.sum(jnp.abs(info), axis=1)

    x1, x2, x3 = readouts
    gvec = lrelu(x1) + lrelu(x2) + lrelu(x3)
    graph_emb = gvec
    gn = gvec / jnp.maximum(jnp.linalg.norm(gvec, axis=-1, keepdims=True), 1e-12)
    gn = lrelu(gn @ lin1_W.T + lin1_b)
    gn = lrelu(gn @ lin2_W.T + lin2_b)
    gn = gn @ lin3_W.T + lin3_b
    return (gn, jnp.mean(node_score), graph_emb)
